# Initial kernel scaffold; baseline (speedup 1.0000x reference)
import jax
import jax.numpy as jnp
from jax.experimental import pallas as pl


def kernel(x, x_pre, edge_index):
    raise NotImplementedError("write your pallas kernel here")


# speedup vs baseline: 7.0356x; 7.0356x over previous
import jax
import jax.numpy as jnp
from jax import lax
from jax.experimental import pallas as pl
from jax.experimental.pallas import tpu as pltpu
from jax.experimental.pallas import tpu_sc as plsc

N = 10000
C = 128
E = 320000
K = 10
ALPHA = 0.1
NC = 2
NS = 16
NW = NC * NS
STRIPE = 632
N_TAB = NS * STRIPE
CHUNK = 128
CPT = 79
E_PAD = NW * CPT * CHUNK

_f32 = jnp.float32


def _sc_segsum_body(init0, init1, tab, src_t, dst_t, es,
                    src_v, dst_v, rows_v, acc_s):
  cid = lax.axis_index("c")
  tid = lax.axis_index("s")
  wid = cid * NS + tid
  stripe = pl.ds(tid * STRIPE, STRIPE)

  pltpu.sync_copy(src_t.at[wid], src_v)
  pltpu.sync_copy(dst_t.at[wid], dst_v)

  for core, init in enumerate((init0, init1)):

    @pl.when(cid == core)
    def _():
      pltpu.sync_copy(init.at[stripe], acc_s.at[stripe])

  plsc.subcore_barrier()

  def chunk_body(j, carry):
    pltpu.sync_copy(tab.at[src_v.at[j]], rows_v)
    pltpu.sync_copy(rows_v, acc_s.at[dst_v.at[j]], add=True)
    return carry

  lax.fori_loop(0, CPT, chunk_body, 0)

  plsc.subcore_barrier()
  pltpu.sync_copy(acc_s.at[stripe], es.at[cid, stripe])


_sc_segsum = pl.kernel(
    _sc_segsum_body,
    out_type=jax.ShapeDtypeStruct((NC, N_TAB, C), _f32),
    mesh=plsc.VectorSubcoreMesh(
        core_axis_name="c", subcore_axis_name="s", num_cores=NC, num_subcores=NS
    ),
    scratch_types=[
        pltpu.VMEM((CPT, CHUNK), jnp.int32),
        pltpu.VMEM((CPT, CHUNK), jnp.int32),
        pltpu.VMEM((CHUNK, C), _f32),
        pltpu.VMEM_SHARED((N_TAB, C), _f32),
    ],
)


def _prep_body(x_ref, deg_ref, dinv_ref, zn_ref):
  deg = deg_ref[0, 0:N, :] + deg_ref[1, 0:N, :]
  dinv = lax.rsqrt(deg)
  dinv_ref[...] = dinv
  zn_ref[0:N, :] = x_ref[...] * dinv
  zn_ref[N:N_TAB, :] = jnp.zeros((N_TAB - N, C), _f32)


_prep = pl.pallas_call(
    _prep_body,
    out_shape=(
        jax.ShapeDtypeStruct((N, C), _f32),
        jax.ShapeDtypeStruct((N_TAB, C), _f32),
    ),
)


def _step_body(x_ref, dinv_ref, es_ref, zn_ref):
  es = es_ref[0, 0:N, :] + es_ref[1, 0:N, :]
  dinv = dinv_ref[...]
  z_new = (1.0 - ALPHA) * dinv * es + ALPHA * x_ref[...]
  zn_ref[0:N, :] = z_new * dinv
  zn_ref[N:N_TAB, :] = jnp.zeros((N_TAB - N, C), _f32)


_step = pl.pallas_call(
    _step_body,
    out_shape=jax.ShapeDtypeStruct((N_TAB, C), _f32),
)


def _final_body(x_ref, x_pre_ref, dinv_ref, es_ref, out_ref):
  es = es_ref[0, 0:N, :] + es_ref[1, 0:N, :]
  x = x_ref[...]
  z_k = (1.0 - ALPHA) * dinv_ref[...] * es + ALPHA * x
  out_ref[...] = 2.0 * x + z_k - x_pre_ref[...]


_final = pl.pallas_call(
    _final_body,
    out_shape=jax.ShapeDtypeStruct((N, C), _f32),
)


def kernel(x, x_pre, edge_index):
  src = edge_index[0]
  dst = edge_index[1]
  pad = E_PAD - E
  padv = jnp.full((pad,), N, jnp.int32)
  src_t = jnp.concatenate([src, padv]).reshape(NW, CPT, CHUNK)
  dst_t = jnp.concatenate([dst, padv]).reshape(NW, CPT, CHUNK)

  ones_tab = jnp.concatenate(
      [jnp.ones((N, C), _f32), jnp.zeros((N_TAB - N, C), _f32)]
  )
  zeros_tab = jnp.zeros((N_TAB, C), _f32)

  es_deg = _sc_segsum(ones_tab, zeros_tab, ones_tab, src_t, dst_t)
  dinv, zn = _prep(x, es_deg)

  for _ in range(K - 1):
    es = _sc_segsum(zn, zeros_tab, zn, src_t, dst_t)
    zn = _step(x, dinv, es)

  es = _sc_segsum(zn, zeros_tab, zn, src_t, dst_t)
  return _final(x, x_pre, dinv, es)
